# baseline (speedup 1.0000x reference)
import functools

import jax
import jax.numpy as jnp
from jax import lax
from jax.experimental import pallas as pl
from jax.experimental.pallas import tpu as pltpu
from jax.experimental.pallas import tpu_sc as plsc

_N, _C, _H, _W = 32, 384, 56, 56
_L = 16
_CT = _C // 128
_GROUPS = _N * _H * (_W // 8)
_GROW = _CT * 8 * 128


def _gather_base():
    perm = jax.random.permutation(jax.random.key(42), _C).astype(jnp.int32)
    return (perm // 128) * 1024 + perm % 128


_NC, _NS = 2, 16
_NW = _NC * _NS
_GPW = _GROUPS // _NW
_GCH = 8
_NCHUNK = _GPW // _GCH
_CHW = _GCH * _GROW
_KB = _C // _L

_mesh = plsc.VectorSubcoreMesh(core_axis_name="c", subcore_axis_name="s")


@functools.partial(
    pl.kernel,
    mesh=_mesh,
    out_type=jax.ShapeDtypeStruct((_GROUPS * _GROW,), jnp.float32),
    compiler_params=pltpu.CompilerParams(
        use_tc_tiling_on_sc=False, needs_layout_passes=False),
    scratch_types=[
        pltpu.VMEM((_C,), jnp.int32),
        pltpu.VMEM((_CHW,), jnp.float32),
        pltpu.VMEM((_CHW,), jnp.float32),
        pltpu.VMEM((_CHW,), jnp.float32),
        pltpu.VMEM((_CHW,), jnp.float32),
        pltpu.SemaphoreType.DMA,
        pltpu.SemaphoreType.DMA,
        pltpu.SemaphoreType.DMA,
        pltpu.SemaphoreType.DMA,
    ],
)
def _shuffle_groups(x_hbm, idx_hbm, out_hbm,
                    idx_v, in0, in1, ou0, ou1, g0, g1, s0, s1):
    ins, ous, gsems, ssems = (in0, in1), (ou0, ou1), (g0, g1), (s0, s1)
    wid = lax.axis_index("s") * _NC + lax.axis_index("c")
    base = wid * _GPW * _GROW
    pltpu.sync_copy(idx_hbm, idx_v)

    def g_copy(v, b):
        return pltpu.make_async_copy(
            x_hbm.at[pl.ds(base + v * _CHW, _CHW)], ins[b], gsems[b])

    def s_copy(v, b):
        return pltpu.make_async_copy(
            ous[b], out_hbm.at[pl.ds(base + v * _CHW, _CHW)], ssems[b])

    ivs = [idx_v[pl.ds(k * _L, _L)] for k in range(_KB)]

    def permute(b):
        @plsc.parallel_loop(0, _GCH * 8, unroll=8)
        def _sweep(q):
            off = (q // 8) * _GROW + (q % 8) * 128
            for k in range(_KB):
                ct, m = divmod(k, 8)
                dst0 = ct * 1024 + m * _L
                g = plsc.load_gather(ins[b], [ivs[k] + off])
                ous[b][pl.ds(dst0 + off, _L)] = g

    g_copy(0, 0).start()

    @pl.loop(0, _NCHUNK, step=2)
    def _(i):
        for b in range(2):
            v = i + b

            @pl.when(v < _NCHUNK)
            def _chunk():
                g_copy(v, b).wait()

                @pl.when(v + 1 < _NCHUNK)
                def _prefetch():
                    g_copy(v + 1, 1 - b).start()

                @pl.when(v >= 2)
                def _drain_prev():
                    s_copy(v - 2, b).wait()
                permute(b)
                s_copy(v, b).start()

    s_copy(_NCHUNK - 2, (_NCHUNK - 2) % 2).wait()
    s_copy(_NCHUNK - 1, (_NCHUNK - 1) % 2).wait()


def kernel(input):
    xp = jnp.transpose(input, (0, 2, 3, 1))
    xp = xp.reshape(_GROUPS, 8, _CT, 128)
    xp = jnp.transpose(xp, (0, 2, 1, 3)).reshape(_GROUPS * _GROW)
    out = _shuffle_groups(xp, _gather_base())
    out = out.reshape(_GROUPS, _CT, 8, 128)
    out = jnp.transpose(out, (0, 2, 1, 3)).reshape(_N, _H, _W, _C)
    return jnp.transpose(out, (0, 3, 1, 2))
